# baseline (speedup 1.0000x reference)
import functools

import jax
import jax.numpy as jnp
from jax import lax
from jax.experimental import pallas as pl
from jax.experimental.pallas import tpu as pltpu
from jax.experimental.pallas import tpu_sc as plsc

N = 10000
E = 320000
D = 128
OUT = 128
K = 2
S = 2
B = 4
DP = B * D

NTILES = 16
PER_TILE = E // NTILES
NB = 8
RPB = 1280
NPAD = NB * RPB
CAP8 = 3072
CH = 32
BLKCH = 48
BPB = CAP8 // (BLKCH * CH)
ROWS_T = RPB // NTILES
LANES = 16
QR = 4


def _hop_sc(xin_builder, in_arrays, psrc, pdst4, pval, zr):
    mesh = plsc.VectorSubcoreMesh(core_axis_name="c", subcore_axis_name="s")

    @functools.partial(
        pl.kernel,
        mesh=mesh,
        out_type=jax.ShapeDtypeStruct((S, QR * NPAD, D), jnp.float32),
        scratch_types=[
            pltpu.VMEM((BLKCH, CH), jnp.int32),
            pltpu.VMEM((BLKCH, QR * CH), jnp.int32),
            pltpu.VMEM((BLKCH, CH), jnp.float32),
            pltpu.VMEM((CH, DP), jnp.float32),
            pltpu.VMEM((CH, DP), jnp.float32),
            pltpu.VMEM((QR * CH, D), jnp.float32),
            pltpu.VMEM((QR * CH, D), jnp.float32),
            pltpu.VMEM_SHARED((QR * RPB, D), jnp.float32),
            pltpu.SemaphoreType.DMA,
            pltpu.SemaphoreType.DMA,
            pltpu.SemaphoreType.DMA,
            pltpu.SemaphoreType.DMA,
        ],
    )
    def k(*refs):
        nin = len(in_arrays)
        ins = refs[:nin]
        src_hbm, dst_hbm, val_hbm, z_hbm, out_hbm = refs[nin:nin + 5]
        (src_v, dst_v, val_v, rows0, rows1, scat0, scat1, acc,
         g0, g1, s0, s1) = refs[nin + 5:]
        rows = (rows0, rows1)
        scat = (scat0, scat1)
        gsem = (g0, g1)
        ssem = (s0, s1)
        c = lax.axis_index("c")
        t = lax.axis_index("s")
        xin = xin_builder(ins, c)

        dnums = lax.GatherDimensionNumbers(
            offset_dims=(), collapsed_slice_dims=(0,), start_index_map=(0,))

        def scale_to(u, cj):
            r = rows[u]
            sc = scat[u]

            def edge(e, carry):
                vv = val_v[cj, pl.ds((e // LANES) * LANES, LANES)]
                lane = jnp.full((LANES,), e % LANES, jnp.int32)
                scale = lax.gather(
                    vv, lane[:, None], dnums, slice_sizes=(1,),
                    mode=lax.GatherScatterMode.PROMISE_IN_BOUNDS)
                for jj in range(DP // LANES):
                    piece = r[e, pl.ds(jj * LANES, LANES)] * scale
                    sc[QR * e + jj // 8,
                       pl.ds((jj % 8) * LANES, LANES)] = piece
                return carry

            lax.fori_loop(0, CH, edge, 0)

        def gissue(u, cj):
            pltpu.async_copy(xin.at[src_v.at[cj]], rows[u], gsem[u])

        def gwait(u):
            pltpu.make_async_copy(xin.at[src_v.at[0]], rows[u],
                                  gsem[u]).wait()

        def sissue(u, cj):
            pltpu.async_copy(scat[u], acc.at[dst4_row(cj)], ssem[u],
                             add=True)

        def dst4_row(cj):
            return dst_v.at[cj]

        def swait(u):
            pltpu.make_async_copy(scat[u], acc.at[dst_v.at[0]],
                                  ssem[u]).wait()

        def pass_body(p, carry):
            pltpu.sync_copy(z_hbm.at[pl.ds(t * QR * ROWS_T, QR * ROWS_T)],
                            acc.at[pl.ds(t * QR * ROWS_T, QR * ROWS_T)])
            plsc.subcore_barrier()

            def block(bi, carry1):
                pltpu.sync_copy(src_hbm.at[c, t, p, bi], src_v)
                pltpu.sync_copy(dst_hbm.at[c, t, p, bi], dst_v)
                pltpu.sync_copy(val_hbm.at[c, t, p, bi], val_v)
                gissue(0, 0)
                gissue(1, 1)
                gwait(0)
                scale_to(0, 0)
                gissue(0, 2)
                sissue(0, 0)
                gwait(1)
                scale_to(1, 1)
                gissue(1, 3)
                sissue(1, 1)

                def pair(i, carry2):
                    cj = 2 * i
                    gwait(0)
                    swait(0)
                    scale_to(0, cj)
                    gissue(0, cj + 2)
                    sissue(0, cj)
                    gwait(1)
                    swait(1)
                    scale_to(1, cj + 1)
                    gissue(1, cj + 3)
                    sissue(1, cj + 1)
                    return carry2

                lax.fori_loop(1, BLKCH // 2 - 1, pair, 0)
                gwait(0)
                swait(0)
                scale_to(0, BLKCH - 2)
                sissue(0, BLKCH - 2)
                gwait(1)
                swait(1)
                scale_to(1, BLKCH - 1)
                sissue(1, BLKCH - 1)
                swait(0)
                swait(1)
                return carry1

            lax.fori_loop(0, BPB, block, 0)
            plsc.subcore_barrier()
            pltpu.sync_copy(
                acc.at[pl.ds(t * QR * ROWS_T, QR * ROWS_T)],
                out_hbm.at[c, pl.ds(p * QR * RPB + t * QR * ROWS_T,
                                    QR * ROWS_T)])
            plsc.subcore_barrier()
            return carry

        lax.fori_loop(0, NB, pass_body, 0)

    return k(*in_arrays, psrc, pdst4, pval, zr)


def _partition(src, dst, val):
    srct = src.reshape(NTILES, PER_TILE)
    dstt = dst.reshape(NTILES, PER_TILE)
    valt = val.reshape(NTILES, PER_TILE)
    q = dstt // RPB
    oh = (q[..., None] == jnp.arange(NB, dtype=q.dtype)).astype(jnp.int32)
    pos_in = jnp.cumsum(oh, axis=1) - 1
    posq = jnp.take_along_axis(pos_in, q[..., None], axis=2)[..., 0]
    posq = jnp.minimum(posq, CAP8 - 1)
    newpos = (jnp.arange(NTILES, dtype=q.dtype)[:, None] * (NB * CAP8)
              + q * CAP8 + posq).ravel()

    def scat(x, dtype):
        z = jnp.zeros((NTILES * NB * CAP8,), dtype)
        z = z.at[newpos].set(x.ravel().astype(dtype), unique_indices=True,
                             mode="promise_in_bounds")
        return z.reshape(NTILES, NB, BPB, BLKCH, CH)

    dl = scat(dstt - q * RPB, jnp.int32)
    dst4 = (dl[..., None] * QR
            + jnp.arange(QR, dtype=jnp.int32)).reshape(
                NTILES, NB, BPB, BLKCH, QR * CH)
    return scat(srct, jnp.int32), dst4, scat(valt, jnp.float32)


def _project_tc(x0, d00, d01, d10, d11, wsum, w1, w2, w4, w5, bias2):
    TN = 1000
    grid = (B, N // TN)
    xspec = pl.BlockSpec((1, TN, D), lambda b, i: (b, i, 0))
    wspec = pl.BlockSpec((D, OUT), lambda b, i: (0, 0))
    bspec = pl.BlockSpec((1, OUT), lambda b, i: (0, 0))

    def body(x0r, ar, br_, cr, dr, w0r, w1r, w2r, w4r, w5r, biasr, outr):
        acc = jnp.dot(x0r[0], w0r[...], preferred_element_type=jnp.float32)
        acc += jnp.dot(ar[0], w1r[...], preferred_element_type=jnp.float32)
        acc += jnp.dot(br_[0], w2r[...], preferred_element_type=jnp.float32)
        acc += jnp.dot(cr[0], w4r[...], preferred_element_type=jnp.float32)
        acc += jnp.dot(dr[0], w5r[...], preferred_element_type=jnp.float32)
        outr[0] = acc + biasr[...]

    return pl.pallas_call(
        body,
        grid=grid,
        in_specs=[xspec, xspec, xspec, xspec, xspec,
                  wspec, wspec, wspec, wspec, wspec, bspec],
        out_specs=pl.BlockSpec((1, TN, OUT), lambda b, i: (b, i, 0)),
        out_shape=jax.ShapeDtypeStruct((B, N, OUT), jnp.float32),
    )(x0, d00, d01, d10, d11, wsum, w1, w2, w4, w5, bias2)


def kernel(inputs, val0, val1, weight, bias, src0, dst0, src1, dst1):
    p0 = _partition(src0, dst0, val0)
    p1 = _partition(src1, dst1, val1)
    psrc = jnp.stack([p0[0], p1[0]])
    pdst4 = jnp.stack([p0[1], p1[1]])
    pval = jnp.stack([p0[2], p1[2]])
    zr = jnp.zeros((QR * RPB, D), jnp.float32)

    x0p = jnp.transpose(inputs, (1, 2, 0)).reshape(N, DP)

    y1q = _hop_sc(lambda ins, c: ins[0], (x0p,), psrc, pdst4, pval, zr)
    y1p = y1q.reshape(S, NPAD, DP)
    y2q = _hop_sc(lambda ins, c: ins[0].at[c], (y1p,), psrc, pdst4, pval, zr)

    def unpack(yp):
        return yp[:, :N, :].reshape(S, N, D, B).transpose(0, 3, 1, 2)

    y1 = unpack(y1p)
    y2 = unpack(y2q.reshape(S, NPAD, DP))

    wb = weight.reshape(S * (K + 1), D, OUT)
    wsum = wb[0] + wb[3]
    return _project_tc(inputs, y1[0], y2[0], y1[1], y2[1],
                       wsum, wb[1], wb[2], wb[4], wb[5], bias.reshape(1, OUT))


# speedup vs baseline: 3.1589x; 3.1589x over previous
import functools

import jax
import jax.numpy as jnp
from jax import lax
from jax.experimental import pallas as pl
from jax.experimental.pallas import tpu as pltpu
from jax.experimental.pallas import tpu_sc as plsc

N = 10000
E = 320000
D = 128
OUT = 128
K = 2
S = 2
B = 4

NTILES = 16
PER_TILE = E // NTILES
CH = 80
NBUF = 4
BLK = 8
NCH = 256
NBLK = NCH // BLK
PAD_PT = NCH * CH
NPAD = 10240
ROWS_T = NPAD // NTILES
LANES = 16


def _diffusion_sc(x0, srcp, dstp, valp, zrows):
    mesh = plsc.VectorSubcoreMesh(core_axis_name="c", subcore_axis_name="s")

    rows_scr = [pltpu.VMEM((CH, D), jnp.float32) for _ in range(NBUF)]
    sem_scr = [pltpu.SemaphoreType.DMA for _ in range(2 * NBUF)]

    @functools.partial(
        pl.kernel,
        mesh=mesh,
        out_type=jax.ShapeDtypeStruct((S, K, B, NPAD, D), jnp.float32),
        scratch_types=[
            pltpu.VMEM((BLK, CH), jnp.int32),
            pltpu.VMEM((BLK, CH), jnp.int32),
            pltpu.VMEM((BLK, CH), jnp.float32),
        ] + rows_scr + [
            pltpu.VMEM_SHARED((NPAD, D), jnp.float32),
        ] + sem_scr,
    )
    def k(x0_hbm, src_hbm, dst_hbm, val_hbm, z_hbm, out_hbm,
          src_v, dst_v, val_v, *scr):
        rows = scr[:NBUF]
        acc = scr[NBUF]
        gsem = scr[NBUF + 1:NBUF + 1 + NBUF]
        ssem = scr[NBUF + 1 + NBUF:]
        c = lax.axis_index("c")
        t = lax.axis_index("s")

        dnums = lax.GatherDimensionNumbers(
            offset_dims=(), collapsed_slice_dims=(0,), start_index_map=(0,))

        def scale_buf(r, cj):
            def edge(e, carry):
                vv = val_v[cj, pl.ds((e // LANES) * LANES, LANES)]
                lane = jnp.full((LANES,), e % LANES, jnp.int32)
                scale = lax.gather(
                    vv, lane[:, None], dnums, slice_sizes=(1,),
                    mode=lax.GatherScatterMode.PROMISE_IN_BOUNDS)
                for j in range(D // LANES):
                    sl = pl.ds(j * LANES, LANES)
                    r[e, sl] = r[e, sl] * scale
                return carry

            lax.fori_loop(0, CH, edge, 0)

        def gissue(xin, u, cj):
            pltpu.async_copy(xin.at[src_v.at[cj]], rows[u], gsem[u])

        def gwait(xin, u):
            pltpu.make_async_copy(xin.at[src_v.at[0]], rows[u], gsem[u]).wait()

        def sissue(u, cj):
            pltpu.async_copy(rows[u], acc.at[dst_v.at[cj]], ssem[u], add=True)

        def swait(u):
            pltpu.make_async_copy(rows[u], acc.at[dst_v.at[0]], ssem[u]).wait()

        def run_task(xin, out_slot):
            pltpu.sync_copy(z_hbm.at[pl.ds(t * ROWS_T, ROWS_T)],
                            acc.at[pl.ds(t * ROWS_T, ROWS_T)])
            plsc.subcore_barrier()

            def block(bi, carry):
                pltpu.sync_copy(src_hbm.at[c, t, pl.ds(bi * BLK, BLK)], src_v)
                pltpu.sync_copy(dst_hbm.at[c, t, pl.ds(bi * BLK, BLK)], dst_v)
                pltpu.sync_copy(val_hbm.at[c, t, pl.ds(bi * BLK, BLK)], val_v)
                for u in range(NBUF):
                    gissue(xin, u, u)
                for u in range(NBUF):
                    gwait(xin, u)
                    scale_buf(rows[u], u)
                    sissue(u, u)
                for u in range(NBUF):
                    swait(u)
                    gissue(xin, u, NBUF + u)
                for u in range(NBUF):
                    gwait(xin, u)
                    scale_buf(rows[u], NBUF + u)
                    sissue(u, NBUF + u)
                for u in range(NBUF):
                    swait(u)
                return carry

            lax.fori_loop(0, NBLK, block, 0)
            plsc.subcore_barrier()
            pltpu.sync_copy(acc.at[pl.ds(t * ROWS_T, ROWS_T)],
                            out_slot.at[pl.ds(t * ROWS_T, ROWS_T)])
            plsc.subcore_barrier()

        def batch_body(b, carry):
            run_task(x0_hbm.at[b], out_hbm.at[c, 0, b])
            run_task(out_hbm.at[c, 0, b], out_hbm.at[c, 1, b])
            return carry

        lax.fori_loop(0, B, batch_body, 0)

    return k(x0, srcp, dstp, valp, zrows)


def _project_tc(x0, d00, d01, d10, d11, wsum, w1, w2, w4, w5, bias2):
    TN = 1000
    grid = (B, N // TN)
    xspec = pl.BlockSpec((1, TN, D), lambda b, i: (b, i, 0))
    wspec = pl.BlockSpec((D, OUT), lambda b, i: (0, 0))
    bspec = pl.BlockSpec((1, OUT), lambda b, i: (0, 0))

    def body(x0r, ar, br_, cr, dr, w0r, w1r, w2r, w4r, w5r, biasr, outr):
        acc = jnp.dot(x0r[0], w0r[...], preferred_element_type=jnp.float32)
        acc += jnp.dot(ar[0], w1r[...], preferred_element_type=jnp.float32)
        acc += jnp.dot(br_[0], w2r[...], preferred_element_type=jnp.float32)
        acc += jnp.dot(cr[0], w4r[...], preferred_element_type=jnp.float32)
        acc += jnp.dot(dr[0], w5r[...], preferred_element_type=jnp.float32)
        outr[0] = acc + biasr[...]

    return pl.pallas_call(
        body,
        grid=grid,
        in_specs=[xspec, xspec, xspec, xspec, xspec,
                  wspec, wspec, wspec, wspec, wspec, bspec],
        out_specs=pl.BlockSpec((1, TN, OUT), lambda b, i: (b, i, 0)),
        out_shape=jax.ShapeDtypeStruct((B, N, OUT), jnp.float32),
    )(x0, d00, d01, d10, d11, wsum, w1, w2, w4, w5, bias2)


def _prep_idx(a):
    a = a.reshape(NTILES, PER_TILE)
    a = jnp.pad(a, ((0, 0), (0, PAD_PT - PER_TILE)))
    return a.reshape(NTILES, NCH, CH)


def _prep_val(v):
    v = v.reshape(NTILES, PER_TILE)
    v = jnp.pad(v, ((0, 0), (0, PAD_PT - PER_TILE)))
    return v.reshape(NTILES, NCH, CH)


def kernel(inputs, val0, val1, weight, bias, src0, dst0, src1, dst1):
    srcp = jnp.stack([_prep_idx(src0), _prep_idx(src1)])
    dstp = jnp.stack([_prep_idx(dst0), _prep_idx(dst1)])
    valp = jnp.stack([_prep_val(val0), _prep_val(val1)])
    zrows = jnp.zeros((NPAD, D), jnp.float32)

    diff = _diffusion_sc(inputs, srcp, dstp, valp, zrows)[:, :, :, :N, :]

    wb = weight.reshape(S * (K + 1), D, OUT)
    wsum = wb[0] + wb[3]
    return _project_tc(inputs, diff[0, 0], diff[0, 1], diff[1, 0], diff[1, 1],
                       wsum, wb[1], wb[2], wb[4], wb[5], bias.reshape(1, OUT))


# speedup vs baseline: 3.7114x; 1.1749x over previous
import functools

import jax
import jax.numpy as jnp
from jax import lax
from jax.experimental import pallas as pl
from jax.experimental.pallas import tpu as pltpu
from jax.experimental.pallas import tpu_sc as plsc

N = 10000
E = 320000
D = 128
OUT = 128
K = 2
S = 2
B = 4

NTILES = 16
PER_TILE = E // NTILES
CH = 128
BLK = 16
NCH = 160
NBLK = NCH // BLK
PAD_PT = NCH * CH
NPAD = 10240
ROWS_T = NPAD // NTILES
LANES = 16


def _diffusion_sc(x0, srcp, dstp, valp, zrows):
    mesh = plsc.VectorSubcoreMesh(core_axis_name="c", subcore_axis_name="s")

    @functools.partial(
        pl.kernel,
        mesh=mesh,
        out_type=jax.ShapeDtypeStruct((S, K, B, NPAD, D), jnp.float32),
        scratch_types=[
            pltpu.VMEM((BLK, CH), jnp.int32),
            pltpu.VMEM((BLK, CH), jnp.int32),
            pltpu.VMEM((BLK, CH), jnp.float32),
            pltpu.VMEM((CH, D), jnp.float32),
            pltpu.VMEM((CH, D), jnp.float32),
            pltpu.VMEM_SHARED((NPAD, D), jnp.float32),
            pltpu.SemaphoreType.DMA,
            pltpu.SemaphoreType.DMA,
            pltpu.SemaphoreType.DMA,
            pltpu.SemaphoreType.DMA,
        ],
    )
    def k(x0_hbm, src_hbm, dst_hbm, val_hbm, z_hbm, out_hbm,
          src_v, dst_v, val_v, rows_a, rows_b, acc,
          sem_ga, sem_gb, sem_sa, sem_sb):
        c = lax.axis_index("c")
        t = lax.axis_index("s")

        dnums = lax.GatherDimensionNumbers(
            offset_dims=(), collapsed_slice_dims=(0,), start_index_map=(0,))

        def scale_buf(rows, cj):
            def grp(g, carry):
                vv = val_v[cj, pl.ds(g * LANES, LANES)]
                for l in range(LANES):
                    scale = lax.gather(
                        vv, jnp.full((LANES, 1), l, jnp.int32),
                        dnums, slice_sizes=(1,),
                        mode=lax.GatherScatterMode.PROMISE_IN_BOUNDS)
                    e = g * LANES + l
                    for j in range(D // LANES):
                        sl = pl.ds(j * LANES, LANES)
                        rows[e, sl] = rows[e, sl] * scale
                return carry

            lax.fori_loop(0, CH // LANES, grp, 0)

        def gather(xin, rows, cj, sem):
            pltpu.async_copy(xin.at[src_v.at[cj]], rows, sem)

        def gwait(xin, rows, sem):
            pltpu.make_async_copy(xin.at[src_v.at[0]], rows, sem).wait()

        def scat(rows, cj, sem):
            pltpu.async_copy(rows, acc.at[dst_v.at[cj]], sem, add=True)

        def swait(rows, sem):
            pltpu.make_async_copy(rows, acc.at[dst_v.at[0]], sem).wait()

        def run_task(xin, out_slot):
            pltpu.sync_copy(z_hbm.at[pl.ds(t * ROWS_T, ROWS_T)],
                            acc.at[pl.ds(t * ROWS_T, ROWS_T)])
            plsc.subcore_barrier()

            def block(bi, carry):
                pltpu.sync_copy(src_hbm.at[c, t, pl.ds(bi * BLK, BLK)], src_v)
                pltpu.sync_copy(dst_hbm.at[c, t, pl.ds(bi * BLK, BLK)], dst_v)
                pltpu.sync_copy(val_hbm.at[c, t, pl.ds(bi * BLK, BLK)], val_v)
                gather(xin, rows_a, 0, sem_ga)
                gather(xin, rows_b, 1, sem_gb)

                def pair(i, carry1):
                    ca = 2 * i
                    gwait(xin, rows_a, sem_ga)
                    scale_buf(rows_a, ca)
                    scat(rows_a, ca, sem_sa)
                    gwait(xin, rows_b, sem_gb)
                    scale_buf(rows_b, ca + 1)
                    scat(rows_b, ca + 1, sem_sb)
                    swait(rows_a, sem_sa)
                    gather(xin, rows_a, ca + 2, sem_ga)
                    swait(rows_b, sem_sb)
                    gather(xin, rows_b, ca + 3, sem_gb)
                    return carry1

                lax.fori_loop(0, BLK // 2 - 1, pair, 0)
                gwait(xin, rows_a, sem_ga)
                scale_buf(rows_a, BLK - 2)
                scat(rows_a, BLK - 2, sem_sa)
                gwait(xin, rows_b, sem_gb)
                scale_buf(rows_b, BLK - 1)
                scat(rows_b, BLK - 1, sem_sb)
                swait(rows_a, sem_sa)
                swait(rows_b, sem_sb)
                return carry

            lax.fori_loop(0, NBLK, block, 0)
            plsc.subcore_barrier()
            pltpu.sync_copy(acc.at[pl.ds(t * ROWS_T, ROWS_T)],
                            out_slot.at[pl.ds(t * ROWS_T, ROWS_T)])
            plsc.subcore_barrier()

        def batch_body(b, carry):
            run_task(x0_hbm.at[b], out_hbm.at[c, 0, b])
            run_task(out_hbm.at[c, 0, b], out_hbm.at[c, 1, b])
            return carry

        lax.fori_loop(0, B, batch_body, 0)

    return k(x0, srcp, dstp, valp, zrows)


def _project_tc(x0, d00, d01, d10, d11, wsum, w1, w2, w4, w5, bias2):
    TN = 1000
    grid = (B, N // TN)
    xspec = pl.BlockSpec((1, TN, D), lambda b, i: (b, i, 0))
    wspec = pl.BlockSpec((D, OUT), lambda b, i: (0, 0))
    bspec = pl.BlockSpec((1, OUT), lambda b, i: (0, 0))

    def body(x0r, ar, br_, cr, dr, w0r, w1r, w2r, w4r, w5r, biasr, outr):
        acc = jnp.dot(x0r[0], w0r[...], preferred_element_type=jnp.float32)
        acc += jnp.dot(ar[0], w1r[...], preferred_element_type=jnp.float32)
        acc += jnp.dot(br_[0], w2r[...], preferred_element_type=jnp.float32)
        acc += jnp.dot(cr[0], w4r[...], preferred_element_type=jnp.float32)
        acc += jnp.dot(dr[0], w5r[...], preferred_element_type=jnp.float32)
        outr[0] = acc + biasr[...]

    return pl.pallas_call(
        body,
        grid=grid,
        in_specs=[xspec, xspec, xspec, xspec, xspec,
                  wspec, wspec, wspec, wspec, wspec, bspec],
        out_specs=pl.BlockSpec((1, TN, OUT), lambda b, i: (b, i, 0)),
        out_shape=jax.ShapeDtypeStruct((B, N, OUT), jnp.float32),
    )(x0, d00, d01, d10, d11, wsum, w1, w2, w4, w5, bias2)


def _prep_idx(a):
    a = a.reshape(NTILES, PER_TILE)
    a = jnp.pad(a, ((0, 0), (0, PAD_PT - PER_TILE)))
    return a.reshape(NTILES, NCH, CH)


def _prep_val(v):
    v = v.reshape(NTILES, PER_TILE)
    v = jnp.pad(v, ((0, 0), (0, PAD_PT - PER_TILE)))
    return v.reshape(NTILES, NCH, CH)


def kernel(inputs, val0, val1, weight, bias, src0, dst0, src1, dst1):
    srcp = jnp.stack([_prep_idx(src0), _prep_idx(src1)])
    dstp = jnp.stack([_prep_idx(dst0), _prep_idx(dst1)])
    valp = jnp.stack([_prep_val(val0), _prep_val(val1)])
    zrows = jnp.zeros((NPAD, D), jnp.float32)

    diff = _diffusion_sc(inputs, srcp, dstp, valp, zrows)[:, :, :, :N, :]

    wb = weight.reshape(S * (K + 1), D, OUT)
    wsum = wb[0] + wb[3]
    return _project_tc(inputs, diff[0, 0], diff[0, 1], diff[1, 0], diff[1, 1],
                       wsum, wb[1], wb[2], wb[4], wb[5], bias.reshape(1, OUT))
